# baseline (speedup 1.0000x reference)
import functools

import jax
import jax.numpy as jnp
from jax import lax
from jax.experimental import pallas as pl
from jax.experimental.pallas import tpu as pltpu
from jax.experimental.pallas import tpu_sc as plsc

N_ROWS = 2097152
NUM_GROUPS = 8192
NUM_CORES = 2
NUM_SUBCORES = 16
LANES = 16
NW = NUM_CORES * NUM_SUBCORES
ROWS_PER_W = N_ROWS // NW
CHUNK = 8192
NCHUNK = ROWS_PER_W // CHUNK

_mesh = plsc.VectorSubcoreMesh(
    core_axis_name="c", subcore_axis_name="s",
    num_cores=NUM_CORES, num_subcores=NUM_SUBCORES)


@functools.partial(
    pl.kernel,
    out_type=(
        jax.ShapeDtypeStruct((NW, NUM_GROUPS), jnp.float32),
        jax.ShapeDtypeStruct((NW, NUM_GROUPS), jnp.float32),
        jax.ShapeDtypeStruct((NW, LANES), jnp.float32),
    ),
    mesh=_mesh,
    compiler_params=pltpu.CompilerParams(needs_layout_passes=False),
    scratch_types=[
        pltpu.VMEM((CHUNK,), jnp.float32),
        pltpu.VMEM((CHUNK,), jnp.float32),
        pltpu.VMEM((CHUNK,), jnp.float32),
        pltpu.VMEM((CHUNK,), jnp.float32),
        pltpu.VMEM((CHUNK + LANES,), jnp.int32),
        pltpu.VMEM((CHUNK + LANES,), jnp.int32),
        pltpu.VMEM((NUM_GROUPS,), jnp.float32),
        pltpu.VMEM((NUM_GROUPS,), jnp.float32),
        pltpu.VMEM((LANES,), jnp.float32),
        pltpu.SemaphoreType.DMA,
        pltpu.SemaphoreType.DMA,
    ],
)
def _sc_pass(pred_hbm, count_hbm, groups_hbm, out_s, out_c, out_p,
             pred_v0, pred_v1, count_v0, count_v1, g_v0, g_v1,
             acc_s, acc_c, pc_v, sem0, sem1):
    wid = lax.axis_index("c") * NUM_SUBCORES + lax.axis_index("s")
    lane = lax.iota(jnp.int32, LANES)
    is15 = lane == (LANES - 1)
    zero16 = jnp.zeros((LANES,), jnp.float32)
    sems = (sem0, sem1)
    pred_bufs = (pred_v0, pred_v1)
    count_bufs = (count_v0, count_v1)
    g_bufs = (g_v0, g_v1)

    @plsc.parallel_loop(0, NUM_GROUPS // LANES, unroll=8)
    def _(i):
        acc_s[pl.ds(i * LANES, LANES)] = zero16
        acc_c[pl.ds(i * LANES, LANES)] = zero16

    def start_copies(ci, slot):
        base = wid * ROWS_PER_W + ci * CHUNK
        sem = sems[slot]
        return (
            pltpu.async_copy(pred_hbm.at[pl.ds(base, CHUNK)], pred_bufs[slot], sem),
            pltpu.async_copy(count_hbm.at[pl.ds(base, CHUNK)], count_bufs[slot], sem),
            pltpu.async_copy(groups_hbm.at[pl.ds(base, CHUNK)],
                             g_bufs[slot].at[pl.ds(0, CHUNK)], sem),
        )

    pc = zero16
    handles = start_copies(0, 0)
    for ci in range(NCHUNK):
        slot = ci % 2
        nxt = None
        if ci + 1 < NCHUNK:
            nxt = start_copies(ci + 1, 1 - slot)
        for h in handles:
            h.wait()
        gs = g_bufs[slot]
        prs = pred_bufs[slot]
        cts = count_bufs[slot]

        @plsc.parallel_loop(0, CHUNK // LANES, unroll=16, carry=pc)
        def pc(i, pc, gs=gs, prs=prs, cts=cts):
            off = i * LANES
            g = gs[pl.ds(off, LANES)]
            gn = gs[pl.ds(off + 1, LANES)]
            pr = prs[pl.ds(off, LANES)]
            ct = cts[pl.ds(off, LANES)]
            e = jnp.exp(pr)
            pc = pc + pr * ct
            ps = plsc.cumsum(e)
            pcnt = plsc.cumsum(ct)
            isend = g != gn
            m_add = isend | is15
            m_sub = isend & jnp.logical_not(is15)
            plsc.addupdate_scatter(acc_s, [g], ps, mask=m_add)
            plsc.addupdate_scatter(acc_s, [gn], -ps, mask=m_sub)
            plsc.addupdate_scatter(acc_c, [g], pcnt, mask=m_add)
            plsc.addupdate_scatter(acc_c, [gn], -pcnt, mask=m_sub)
            return pc

        handles = nxt
    pc_v[...] = pc
    pltpu.sync_copy(acc_s, out_s.at[wid])
    pltpu.sync_copy(acc_c, out_c.at[wid])
    pltpu.sync_copy(pc_v, out_p.at[wid])


def _tc_body(s_ref, c_ref, p_ref, o_ref):
    s = jnp.sum(s_ref[...], axis=0)
    c = jnp.sum(c_ref[...], axis=0)
    nonempty = s > 0.0
    contrib = jnp.where(nonempty, jnp.log(jnp.where(nonempty, s, 1.0)) * c, 0.0)
    o_ref[0, 0] = (jnp.sum(contrib) - jnp.sum(p_ref[...])) * (1.0 / N_ROWS)


_tc_finish = pl.pallas_call(
    _tc_body,
    out_shape=jax.ShapeDtypeStruct((1, 1), jnp.float32),
    out_specs=pl.BlockSpec(memory_space=pltpu.SMEM),
)


def kernel(pred, count, groups):
    pred1 = pred.reshape(N_ROWS)
    count1 = count.reshape(N_ROWS)
    out_s, out_c, out_p = _sc_pass(pred1, count1, groups)
    return _tc_finish(out_s, out_c, out_p)[0, 0]


# speedup vs baseline: 1.0441x; 1.0441x over previous
import functools

import jax
import jax.numpy as jnp
from jax import lax
from jax.experimental import pallas as pl
from jax.experimental.pallas import tpu as pltpu
from jax.experimental.pallas import tpu_sc as plsc

N_ROWS = 2097152
NUM_GROUPS = 8192
NUM_CORES = 2
NUM_SUBCORES = 16
LANES = 16
NW = NUM_CORES * NUM_SUBCORES
ROWS_PER_W = N_ROWS // NW
CHUNK = 8192
NCHUNK = ROWS_PER_W // CHUNK

_mesh = plsc.VectorSubcoreMesh(
    core_axis_name="c", subcore_axis_name="s",
    num_cores=NUM_CORES, num_subcores=NUM_SUBCORES)


@functools.partial(
    pl.kernel,
    out_type=(
        jax.ShapeDtypeStruct((NW, NUM_GROUPS), jnp.float32),
        jax.ShapeDtypeStruct((NW, NUM_GROUPS), jnp.float32),
        jax.ShapeDtypeStruct((NW, LANES), jnp.float32),
    ),
    mesh=_mesh,
    compiler_params=pltpu.CompilerParams(needs_layout_passes=False),
    scratch_types=[
        pltpu.VMEM((CHUNK,), jnp.float32),
        pltpu.VMEM((CHUNK,), jnp.float32),
        pltpu.VMEM((CHUNK,), jnp.float32),
        pltpu.VMEM((CHUNK,), jnp.float32),
        pltpu.VMEM((CHUNK + LANES,), jnp.int32),
        pltpu.VMEM((CHUNK + LANES,), jnp.int32),
        pltpu.VMEM((NUM_GROUPS,), jnp.float32),
        pltpu.VMEM((NUM_GROUPS,), jnp.float32),
        pltpu.VMEM((LANES,), jnp.float32),
        pltpu.SemaphoreType.DMA,
        pltpu.SemaphoreType.DMA,
    ],
)
def _sc_pass(pred_hbm, count_hbm, groups_hbm, out_s, out_c, out_p,
             pred_v0, pred_v1, count_v0, count_v1, g_v0, g_v1,
             acc_s, acc_c, pc_v, sem0, sem1):
    wid = lax.axis_index("c") * NUM_SUBCORES + lax.axis_index("s")
    lane = lax.iota(jnp.int32, LANES)
    is15 = lane == (LANES - 1)
    zero16 = jnp.zeros((LANES,), jnp.float32)
    sems = (sem0, sem1)
    pred_bufs = (pred_v0, pred_v1)
    count_bufs = (count_v0, count_v1)
    g_bufs = (g_v0, g_v1)

    @plsc.parallel_loop(0, NUM_GROUPS // LANES, unroll=8)
    def _(i):
        acc_s[pl.ds(i * LANES, LANES)] = zero16
        acc_c[pl.ds(i * LANES, LANES)] = zero16

    def start_copies(ci, slot):
        base = wid * ROWS_PER_W + ci * CHUNK
        sem = sems[slot]
        return (
            pltpu.async_copy(pred_hbm.at[pl.ds(base, CHUNK)], pred_bufs[slot], sem),
            pltpu.async_copy(count_hbm.at[pl.ds(base, CHUNK)], count_bufs[slot], sem),
            pltpu.async_copy(groups_hbm.at[pl.ds(base, CHUNK)],
                             g_bufs[slot].at[pl.ds(0, CHUNK)], sem),
        )

    pc = zero16
    handles = start_copies(0, 0)
    for ci in range(NCHUNK):
        slot = ci % 2
        nxt = None
        if ci + 1 < NCHUNK:
            nxt = start_copies(ci + 1, 1 - slot)
        for h in handles:
            h.wait()
        gs = g_bufs[slot]
        prs = pred_bufs[slot]
        cts = count_bufs[slot]

        @plsc.parallel_loop(0, CHUNK // LANES, unroll=4, carry=pc)
        def pc(i, pc, gs=gs, prs=prs, cts=cts):
            off = i * LANES
            g = gs[pl.ds(off, LANES)]
            gn = gs[pl.ds(off + 1, LANES)]
            pr = prs[pl.ds(off, LANES)]
            ct = cts[pl.ds(off, LANES)]
            e = jnp.exp(pr)
            pc = pc + pr * ct
            ps = plsc.cumsum(e)
            pcnt = plsc.cumsum(ct)
            isend = g != gn
            m_add = isend | is15
            m_sub = isend & jnp.logical_not(is15)
            plsc.addupdate_scatter(acc_s, [g], ps, mask=m_add)
            plsc.addupdate_scatter(acc_s, [gn], -ps, mask=m_sub)
            plsc.addupdate_scatter(acc_c, [g], pcnt, mask=m_add)
            plsc.addupdate_scatter(acc_c, [gn], -pcnt, mask=m_sub)
            return pc

        handles = nxt
    pc_v[...] = pc
    pltpu.sync_copy(acc_s, out_s.at[wid])
    pltpu.sync_copy(acc_c, out_c.at[wid])
    pltpu.sync_copy(pc_v, out_p.at[wid])


def _tc_body(s_ref, c_ref, p_ref, o_ref):
    s = jnp.sum(s_ref[...], axis=0)
    c = jnp.sum(c_ref[...], axis=0)
    nonempty = s > 0.0
    contrib = jnp.where(nonempty, jnp.log(jnp.where(nonempty, s, 1.0)) * c, 0.0)
    o_ref[0, 0] = (jnp.sum(contrib) - jnp.sum(p_ref[...])) * (1.0 / N_ROWS)


_tc_finish = pl.pallas_call(
    _tc_body,
    out_shape=jax.ShapeDtypeStruct((1, 1), jnp.float32),
    out_specs=pl.BlockSpec(memory_space=pltpu.SMEM),
)


def kernel(pred, count, groups):
    pred1 = pred.reshape(N_ROWS)
    count1 = count.reshape(N_ROWS)
    out_s, out_c, out_p = _sc_pass(pred1, count1, groups)
    return _tc_finish(out_s, out_c, out_p)[0, 0]


# speedup vs baseline: 1.0925x; 1.0464x over previous
import functools

import jax
import jax.numpy as jnp
from jax import lax
from jax.experimental import pallas as pl
from jax.experimental.pallas import tpu as pltpu
from jax.experimental.pallas import tpu_sc as plsc

N_ROWS = 2097152
NUM_GROUPS = 8192
NUM_CORES = 2
NUM_SUBCORES = 16
LANES = 16
NW = NUM_CORES * NUM_SUBCORES
ROWS_PER_W = N_ROWS // NW
CHUNK = 8192
NCHUNK = ROWS_PER_W // CHUNK

_mesh = plsc.VectorSubcoreMesh(
    core_axis_name="c", subcore_axis_name="s",
    num_cores=NUM_CORES, num_subcores=NUM_SUBCORES)


@functools.partial(
    pl.kernel,
    out_type=(
        jax.ShapeDtypeStruct((NW, NUM_GROUPS), jnp.float32),
        jax.ShapeDtypeStruct((NW, NUM_GROUPS), jnp.float32),
        jax.ShapeDtypeStruct((NW, LANES), jnp.float32),
    ),
    mesh=_mesh,
    compiler_params=pltpu.CompilerParams(needs_layout_passes=False),
    scratch_types=[
        pltpu.VMEM((CHUNK,), jnp.float32),
        pltpu.VMEM((CHUNK,), jnp.float32),
        pltpu.VMEM((CHUNK,), jnp.float32),
        pltpu.VMEM((CHUNK,), jnp.float32),
        pltpu.VMEM((CHUNK + LANES,), jnp.int32),
        pltpu.VMEM((CHUNK + LANES,), jnp.int32),
        pltpu.VMEM((NUM_GROUPS,), jnp.float32),
        pltpu.VMEM((NUM_GROUPS,), jnp.float32),
        pltpu.VMEM((LANES,), jnp.float32),
        pltpu.SemaphoreType.DMA,
        pltpu.SemaphoreType.DMA,
    ],
)
def _sc_pass(pred_hbm, count_hbm, groups_hbm, out_s, out_c, out_p,
             pred_v0, pred_v1, count_v0, count_v1, g_v0, g_v1,
             acc_s, acc_c, pc_v, sem0, sem1):
    wid = lax.axis_index("c") * NUM_SUBCORES + lax.axis_index("s")
    lane = lax.iota(jnp.int32, LANES)
    is15 = lane == (LANES - 1)
    shift_idx = jnp.minimum(lane + 1, LANES - 1)
    zero16 = jnp.zeros((LANES,), jnp.float32)
    sems = (sem0, sem1)
    pred_bufs = (pred_v0, pred_v1)
    count_bufs = (count_v0, count_v1)
    g_bufs = (g_v0, g_v1)

    @plsc.parallel_loop(0, NUM_GROUPS // LANES, unroll=8)
    def _(i):
        acc_s[pl.ds(i * LANES, LANES)] = zero16
        acc_c[pl.ds(i * LANES, LANES)] = zero16

    def start_copies(ci, slot):
        base = wid * ROWS_PER_W + ci * CHUNK
        sem = sems[slot]
        return (
            pltpu.async_copy(pred_hbm.at[pl.ds(base, CHUNK)], pred_bufs[slot], sem),
            pltpu.async_copy(count_hbm.at[pl.ds(base, CHUNK)], count_bufs[slot], sem),
            pltpu.async_copy(groups_hbm.at[pl.ds(base, CHUNK)],
                             g_bufs[slot].at[pl.ds(0, CHUNK)], sem),
        )

    pc = zero16
    handles = start_copies(0, 0)
    for ci in range(NCHUNK):
        slot = ci % 2
        nxt = None
        if ci + 1 < NCHUNK:
            nxt = start_copies(ci + 1, 1 - slot)
        for h in handles:
            h.wait()
        gs = g_bufs[slot]
        prs = pred_bufs[slot]
        cts = count_bufs[slot]

        @plsc.parallel_loop(0, CHUNK // LANES, unroll=4, carry=pc)
        def pc(i, pc, gs=gs, prs=prs, cts=cts):
            off = i * LANES
            g = gs[pl.ds(off, LANES)]
            gn = lax.gather(
                g, shift_idx[:, None],
                lax.GatherDimensionNumbers(
                    offset_dims=(), collapsed_slice_dims=(0,),
                    start_index_map=(0,)),
                (1,), mode=lax.GatherScatterMode.PROMISE_IN_BOUNDS)
            pr = prs[pl.ds(off, LANES)]
            ct = cts[pl.ds(off, LANES)]
            e = jnp.exp(pr)
            pc = pc + pr * ct
            ps = plsc.cumsum(e)
            pcnt = plsc.cumsum(ct)
            isend = g != gn
            m_add = isend | is15
            m_sub = isend & jnp.logical_not(is15)
            plsc.addupdate_scatter(acc_s, [g], ps, mask=m_add)
            plsc.addupdate_scatter(acc_s, [gn], -ps, mask=m_sub)
            plsc.addupdate_scatter(acc_c, [g], pcnt, mask=m_add)
            plsc.addupdate_scatter(acc_c, [gn], -pcnt, mask=m_sub)
            return pc

        handles = nxt
    pc_v[...] = pc
    pltpu.sync_copy(acc_s, out_s.at[wid])
    pltpu.sync_copy(acc_c, out_c.at[wid])
    pltpu.sync_copy(pc_v, out_p.at[wid])


def _tc_body(s_ref, c_ref, p_ref, o_ref):
    s = jnp.sum(s_ref[...], axis=0)
    c = jnp.sum(c_ref[...], axis=0)
    nonempty = s > 0.0
    contrib = jnp.where(nonempty, jnp.log(jnp.where(nonempty, s, 1.0)) * c, 0.0)
    o_ref[0, 0] = (jnp.sum(contrib) - jnp.sum(p_ref[...])) * (1.0 / N_ROWS)


_tc_finish = pl.pallas_call(
    _tc_body,
    out_shape=jax.ShapeDtypeStruct((1, 1), jnp.float32),
    out_specs=pl.BlockSpec(memory_space=pltpu.SMEM),
)


def kernel(pred, count, groups):
    pred1 = pred.reshape(N_ROWS)
    count1 = count.reshape(N_ROWS)
    out_s, out_c, out_p = _sc_pass(pred1, count1, groups)
    return _tc_finish(out_s, out_c, out_p)[0, 0]
